# baseline (speedup 1.0000x reference)
import functools

import jax
import jax.numpy as jnp
from jax import lax
from jax.experimental import pallas as pl
from jax.experimental.pallas import tpu as pltpu
from jax.experimental.pallas import tpu_sc as plsc

_K = 64
_ROWS = 64
_N = 32768
_L = 16
_NV = _N // _L
_U = 8
_NBLK = 64
_BV = _NV // _NBLK
_NC = 2
_NS = 16
_NW = _NC * _NS
_RPW = _ROWS // _NW
_NB = 256


def _key_of(xv):
    xi = lax.bitcast_convert_type(xv, jnp.int32)
    sa = lax.shift_right_arithmetic(xi, jnp.int32(31))
    return lax.bitwise_xor(xi, lax.bitwise_and(sa, jnp.int32(0x7FFFFFFF)))


def _scan_hist(hist_v, need):
    nch = _NB // _L
    iota = lax.iota(jnp.int32, _L)

    def coarse(t0, carry):
        s, jsel, ssel = carry
        for u in range(_U):
            j = (nch - 1) - (t0 * _U + u)
            tot = jnp.sum(hist_v[pl.ds(j * _L, _L)])
            cond = jnp.logical_and(s < need, s + tot >= need)
            jsel = jnp.where(cond, j, jsel)
            ssel = jnp.where(cond, s, ssel)
            s = s + tot
        return s, jsel, ssel

    _, jsel, ssel = lax.fori_loop(
        0, nch // _U, coarse,
        (jnp.int32(0), jnp.int32(0), jnp.int32(0)), unroll=False)

    h = hist_v[pl.ds(jsel * _L, _L)]
    rev = lax.rev(h, (0,))
    csum = plsc.cumsum(rev)
    s_lane = ssel + csum
    bin_lane = (jsel * _L + (_L - 1)) - iota
    cond = s_lane >= need
    cand_c = jnp.where(cond, lax.shift_left(bin_lane, jnp.int32(16)) +
                       (s_lane - rev), jnp.int32(-1))
    best_c = jnp.max(cand_c)
    bsel = lax.shift_right_arithmetic(best_c, jnp.int32(16))
    csel = lax.bitwise_and(best_c, jnp.int32(0xFFFF))
    return bsel, csel


def _topk_row(row_v, cand_v, idx_v, hist_v):
    iota = lax.iota(jnp.int32, _L)
    ones = jnp.full((_L,), 1, jnp.int32)
    zeros_i = jnp.zeros((_L,), jnp.int32)
    zeros_f = jnp.zeros((_L,), jnp.float32)
    k8 = jnp.int32(0xFF)

    def blockmax(b, t0):
        mx = row_v[pl.ds(b * (_BV * _L), _L)]
        for u in range(1, _BV):
            mx = jnp.maximum(mx, row_v[pl.ds(b * (_BV * _L) + u * _L, _L)])
        return jnp.minimum(t0, jnp.max(mx))
    t0 = lax.fori_loop(0, _NBLK, blockmax, jnp.float32(float("inf")),
                       unroll=False)

    def filt(i, off_v):
        for u in range(_U):
            iv = i * _U + u
            xv = row_v[pl.ds(iv * _L, _L)]
            m_c = xv >= t0
            row_v[pl.ds(iv * _L, _L)] = jnp.where(m_c, xv, zeros_f)
            eq1 = jnp.where(m_c, ones, zeros_i)
            pref = plsc.cumsum(eq1)
            pos = (off_v + pref) - 1
            plsc.store_scatter(cand_v, [pos], xv, mask=m_c)
            plsc.store_scatter(idx_v, [pos], jnp.int32(iv * _L) + iota,
                               mask=m_c)
            off_v = off_v + plsc.all_reduce_population_count(m_c)
        return off_v
    off_v = lax.fori_loop(0, _NV // _U, filt, zeros_i, unroll=False)
    m = off_v[0]
    nvc = lax.shift_right_arithmetic(m + jnp.int32(_L - 1), jnp.int32(4))

    def level(shift, prefix, need):
        def z(i, _):
            hist_v[pl.ds(i * _L, _L)] = zeros_i
            return 0
        lax.fori_loop(0, _NB // _L, z, 0, unroll=False)

        def h(i, _):
            key = _key_of(cand_v[pl.ds(i * _L, _L)])
            gmask = (i * _L + iota) < m
            if shift == 24:
                mk = gmask
                b = lax.shift_right_arithmetic(key, jnp.int32(24)) + \
                    jnp.int32(128)
            else:
                hi = lax.shift_right_arithmetic(key, jnp.int32(shift + 8))
                mk = jnp.logical_and(gmask, hi == prefix)
                b = lax.bitwise_and(
                    lax.shift_right_arithmetic(key, jnp.int32(shift)), k8)
            plsc.addupdate_scatter(hist_v, [b], ones, mask=mk)
            return 0
        lax.fori_loop(0, nvc, h, 0, unroll=False)
        return _scan_hist(hist_v, need)

    b1, c1 = level(24, None, jnp.int32(_K))
    t8 = b1 - jnp.int32(128)
    need2 = jnp.int32(_K) - c1
    b2, c2 = level(16, t8, need2)
    t16 = lax.bitwise_or(lax.shift_left(t8, jnp.int32(8)), b2)
    need3 = need2 - c2
    b3, c3 = level(8, t16, need3)
    t24 = lax.bitwise_or(lax.shift_left(t16, jnp.int32(8)), b3)
    need4 = need3 - c3
    b4, c4 = level(0, t24, need4)
    thresh = lax.bitwise_or(lax.shift_left(t24, jnp.int32(8)), b4)
    need_eq = need4 - c4

    def drop(i, e):
        kv = cand_v[pl.ds(i * _L, _L)]
        pv = idx_v[pl.ds(i * _L, _L)]
        key = _key_of(kv)
        gmask = (i * _L + iota) < m
        m_gt = jnp.logical_and(gmask, key > thresh)
        m_eq = jnp.logical_and(gmask, key == thresh)
        eq1 = jnp.where(m_eq, ones, zeros_i)
        pref = plsc.cumsum(eq1)
        keep = jnp.logical_or(m_gt,
                              jnp.logical_and(m_eq, (e + pref) <= need_eq))
        kill = jnp.logical_and(gmask, jnp.logical_not(keep))
        plsc.store_scatter(row_v, [pv], zeros_f, mask=kill)
        return e + jnp.sum(eq1)
    lax.fori_loop(0, nvc, drop, jnp.int32(0), unroll=False)


@functools.partial(
    pl.kernel,
    out_type=jax.ShapeDtypeStruct((_ROWS, _N), jnp.float32),
    mesh=plsc.VectorSubcoreMesh(core_axis_name="c", subcore_axis_name="s"),
    compiler_params=pltpu.CompilerParams(needs_layout_passes=False),
    scratch_types=[
        pltpu.VMEM((_N,), jnp.float32),
        pltpu.VMEM((_N,), jnp.float32),
        pltpu.VMEM((_N,), jnp.int32),
        pltpu.VMEM((_NB,), jnp.int32),
    ],
)
def _topk_sc(x_hbm, out_hbm, row_v, cand_v, idx_v, hist_v):
    wid = lax.axis_index("s") * _NC + lax.axis_index("c")
    for r in range(_RPW):
        row = wid * _RPW + r
        pltpu.sync_copy(x_hbm.at[row], row_v)
        _topk_row(row_v, cand_v, idx_v, hist_v)
        pltpu.sync_copy(row_v, out_hbm.at[row])


@jax.jit
def kernel(x):
    return _topk_sc(x)


# speedup vs baseline: 1.2221x; 1.2221x over previous
import functools

import jax
import jax.numpy as jnp
from jax import lax
from jax.experimental import pallas as pl
from jax.experimental.pallas import tpu as pltpu
from jax.experimental.pallas import tpu_sc as plsc

_K = 64
_ROWS = 64
_N = 32768
_L = 16
_NV = _N // _L
_U = 8
_NBLK = 64
_BV = _NV // _NBLK
_NC = 2
_NS = 16
_NW = _NC * _NS
_RPW = _ROWS // _NW
_NB = 256


def _key_of(xv):
    xi = lax.bitcast_convert_type(xv, jnp.int32)
    sa = lax.shift_right_arithmetic(xi, jnp.int32(31))
    return lax.bitwise_xor(xi, lax.bitwise_and(sa, jnp.int32(0x7FFFFFFF)))


def _scan_hist(hist_v, need):
    nch = _NB // _L
    iota = lax.iota(jnp.int32, _L)

    def coarse(t0, carry):
        s, jsel, ssel = carry
        for u in range(_U):
            j = (nch - 1) - (t0 * _U + u)
            tot = jnp.sum(hist_v[pl.ds(j * _L, _L)])
            cond = jnp.logical_and(s < need, s + tot >= need)
            jsel = jnp.where(cond, j, jsel)
            ssel = jnp.where(cond, s, ssel)
            s = s + tot
        return s, jsel, ssel

    _, jsel, ssel = lax.fori_loop(
        0, nch // _U, coarse,
        (jnp.int32(0), jnp.int32(0), jnp.int32(0)), unroll=False)

    h = hist_v[pl.ds(jsel * _L, _L)]
    rev = lax.rev(h, (0,))
    csum = plsc.cumsum(rev)
    s_lane = ssel + csum
    bin_lane = (jsel * _L + (_L - 1)) - iota
    cond = s_lane >= need
    cand_c = jnp.where(cond, lax.shift_left(bin_lane, jnp.int32(16)) +
                       (s_lane - rev), jnp.int32(-1))
    best_c = jnp.max(cand_c)
    bsel = lax.shift_right_arithmetic(best_c, jnp.int32(16))
    csel = lax.bitwise_and(best_c, jnp.int32(0xFFFF))
    return bsel, csel


def _topk_row(row_v, cand_v, idx_v, hist_v):
    iota = lax.iota(jnp.int32, _L)
    ones = jnp.full((_L,), 1, jnp.int32)
    zeros_i = jnp.zeros((_L,), jnp.int32)
    zeros_f = jnp.zeros((_L,), jnp.float32)
    k8 = jnp.int32(0xFF)

    def blockmax(b, t0):
        mx = row_v[pl.ds(b * (_BV * _L), _L)]
        for u in range(1, _BV):
            mx = jnp.maximum(mx, row_v[pl.ds(b * (_BV * _L) + u * _L, _L)])
        return jnp.minimum(t0, jnp.max(mx))
    t0 = lax.fori_loop(0, _NBLK, blockmax, jnp.float32(float("inf")),
                       unroll=False)

    def filt(i, off):
        for u in range(_U):
            iv = i * _U + u
            xv = row_v[pl.ds(iv * _L, _L)]
            m_c = xv >= t0
            row_v[pl.ds(iv * _L, _L)] = jnp.where(m_c, xv, zeros_f)
            plsc.store_compressed(cand_v.at[pl.ds(off, _L)], xv, mask=m_c)
            plsc.store_compressed(idx_v.at[pl.ds(off, _L)],
                                  jnp.int32(iv * _L) + iota, mask=m_c)
            pc = plsc.all_reduce_population_count(m_c)
            off = off + pc[0]
        return off
    m = lax.fori_loop(0, _NV // _U, filt, jnp.int32(0), unroll=False)
    nvc = lax.shift_right_arithmetic(m + jnp.int32(_L - 1), jnp.int32(4))

    def level(shift, prefix, need):
        def z(i, _):
            hist_v[pl.ds(i * _L, _L)] = zeros_i
            return 0
        lax.fori_loop(0, _NB // _L, z, 0, unroll=False)

        def h(i, _):
            key = _key_of(cand_v[pl.ds(i * _L, _L)])
            gmask = (i * _L + iota) < m
            if shift == 24:
                mk = gmask
                b = lax.shift_right_arithmetic(key, jnp.int32(24)) + \
                    jnp.int32(128)
            else:
                hi = lax.shift_right_arithmetic(key, jnp.int32(shift + 8))
                mk = jnp.logical_and(gmask, hi == prefix)
                b = lax.bitwise_and(
                    lax.shift_right_arithmetic(key, jnp.int32(shift)), k8)
            plsc.addupdate_scatter(hist_v, [b], ones, mask=mk)
            return 0
        lax.fori_loop(0, nvc, h, 0, unroll=False)
        return _scan_hist(hist_v, need)

    b1, c1 = level(24, None, jnp.int32(_K))
    t8 = b1 - jnp.int32(128)
    need2 = jnp.int32(_K) - c1
    b2, c2 = level(16, t8, need2)
    t16 = lax.bitwise_or(lax.shift_left(t8, jnp.int32(8)), b2)
    need3 = need2 - c2
    b3, c3 = level(8, t16, need3)
    t24 = lax.bitwise_or(lax.shift_left(t16, jnp.int32(8)), b3)
    need4 = need3 - c3
    b4, c4 = level(0, t24, need4)
    thresh = lax.bitwise_or(lax.shift_left(t24, jnp.int32(8)), b4)
    need_eq = need4 - c4

    def drop(i, e):
        kv = cand_v[pl.ds(i * _L, _L)]
        pv = idx_v[pl.ds(i * _L, _L)]
        key = _key_of(kv)
        gmask = (i * _L + iota) < m
        m_gt = jnp.logical_and(gmask, key > thresh)
        m_eq = jnp.logical_and(gmask, key == thresh)
        eq1 = jnp.where(m_eq, ones, zeros_i)
        pref = plsc.cumsum(eq1)
        keep = jnp.logical_or(m_gt,
                              jnp.logical_and(m_eq, (e + pref) <= need_eq))
        kill = jnp.logical_and(gmask, jnp.logical_not(keep))
        plsc.store_scatter(row_v, [pv], zeros_f, mask=kill)
        return e + jnp.sum(eq1)
    lax.fori_loop(0, nvc, drop, jnp.int32(0), unroll=False)


@functools.partial(
    pl.kernel,
    out_type=jax.ShapeDtypeStruct((_ROWS, _N), jnp.float32),
    mesh=plsc.VectorSubcoreMesh(core_axis_name="c", subcore_axis_name="s"),
    compiler_params=pltpu.CompilerParams(needs_layout_passes=False),
    scratch_types=[
        pltpu.VMEM((_N,), jnp.float32),
        pltpu.VMEM((_N,), jnp.float32),
        pltpu.VMEM((_N,), jnp.int32),
        pltpu.VMEM((_NB,), jnp.int32),
    ],
)
def _topk_sc(x_hbm, out_hbm, row_v, cand_v, idx_v, hist_v):
    wid = lax.axis_index("s") * _NC + lax.axis_index("c")
    for r in range(_RPW):
        row = wid * _RPW + r
        pltpu.sync_copy(x_hbm.at[row], row_v)
        _topk_row(row_v, cand_v, idx_v, hist_v)
        pltpu.sync_copy(row_v, out_hbm.at[row])


@jax.jit
def kernel(x):
    return _topk_sc(x)


# speedup vs baseline: 1.2474x; 1.0207x over previous
import functools

import jax
import jax.numpy as jnp
from jax import lax
from jax.experimental import pallas as pl
from jax.experimental.pallas import tpu as pltpu
from jax.experimental.pallas import tpu_sc as plsc

_K = 64
_ROWS = 64
_N = 32768
_L = 16
_NV = _N // _L
_NQ = 4
_QV = _NV // _NQ
_QN = _N // _NQ
_UB = 2
_U = 8
_NBLK = 64
_BV = _NV // _NBLK
_NC = 2
_NS = 16
_NW = _NC * _NS
_RPW = _ROWS // _NW
_NB = 256


def _key_of(xv):
    xi = lax.bitcast_convert_type(xv, jnp.int32)
    sa = lax.shift_right_arithmetic(xi, jnp.int32(31))
    return lax.bitwise_xor(xi, lax.bitwise_and(sa, jnp.int32(0x7FFFFFFF)))


def _scan_hist(hist_v, need):
    nch = _NB // _L
    iota = lax.iota(jnp.int32, _L)

    def coarse(t, carry):
        s, jsel, ssel = carry
        for u in range(_U):
            j = (nch - 1) - (t * _U + u)
            tot = jnp.sum(hist_v[pl.ds(j * _L, _L)])
            cond = jnp.logical_and(s < need, s + tot >= need)
            jsel = jnp.where(cond, j, jsel)
            ssel = jnp.where(cond, s, ssel)
            s = s + tot
        return s, jsel, ssel

    _, jsel, ssel = lax.fori_loop(
        0, nch // _U, coarse,
        (jnp.int32(0), jnp.int32(0), jnp.int32(0)), unroll=False)

    h = hist_v[pl.ds(jsel * _L, _L)]
    rev = lax.rev(h, (0,))
    csum = plsc.cumsum(rev)
    s_lane = ssel + csum
    bin_lane = (jsel * _L + (_L - 1)) - iota
    cond = s_lane >= need
    cand_c = jnp.where(cond, lax.shift_left(bin_lane, jnp.int32(16)) +
                       (s_lane - rev), jnp.int32(-1))
    cand_h = jnp.where(cond, lax.shift_left(bin_lane, jnp.int32(16)) + rev,
                       jnp.int32(-1))
    best_c = jnp.max(cand_c)
    best_h = jnp.max(cand_h)
    bsel = lax.shift_right_arithmetic(best_c, jnp.int32(16))
    csel = lax.bitwise_and(best_c, jnp.int32(0xFFFF))
    hsel = lax.bitwise_and(best_h, jnp.int32(0xFFFF))
    return bsel, csel, hsel


def _topk_row(row_v, cand_v, hist_v):
    iota = lax.iota(jnp.int32, _L)
    ones = jnp.full((_L,), 1, jnp.int32)
    zeros_i = jnp.zeros((_L,), jnp.int32)
    zeros_f = jnp.zeros((_L,), jnp.float32)
    neg_inf = jnp.full((_L,), float("-inf"), jnp.float32)
    k8 = jnp.int32(0xFF)

    def blockmax(b, t0):
        mx = row_v[pl.ds(b * (_BV * _L), _L)]
        for u in range(1, _BV):
            mx = jnp.maximum(mx, row_v[pl.ds(b * (_BV * _L) + u * _L, _L)])
        return jnp.minimum(t0, jnp.max(mx))
    t0 = lax.fori_loop(0, _NBLK, blockmax, jnp.float32(float("inf")),
                       unroll=False)

    def filt(i, offs):
        offs = list(offs)
        for u in range(_UB):
            iv0 = i * _UB + u
            for q in range(_NQ):
                iv = q * _QV + iv0
                xv = row_v[pl.ds(iv * _L, _L)]
                m_c = xv >= t0
                plsc.store_compressed(cand_v.at[pl.ds(offs[q], _L)], xv,
                                      mask=m_c)
                pc = plsc.all_reduce_population_count(m_c)
                offs[q] = offs[q] + pc[0]
        return tuple(offs)
    offs = lax.fori_loop(
        0, _QV // _UB, filt,
        tuple(jnp.int32(q * _QN) for q in range(_NQ)), unroll=False)
    mq = [offs[q] - jnp.int32(q * _QN) for q in range(_NQ)]
    nvcq = [lax.shift_right_arithmetic(mq[q] + jnp.int32(_L - 1),
                                       jnp.int32(4)) for q in range(_NQ)]

    def level(shift, prefix, need):
        def z(i, _):
            hist_v[pl.ds(i * _L, _L)] = zeros_i
            return 0
        lax.fori_loop(0, _NB // _L, z, 0, unroll=False)

        for q in range(_NQ):
            def h(i, _, q=q):
                key = _key_of(cand_v[pl.ds(q * _QN + i * _L, _L)])
                gmask = (i * _L + iota) < mq[q]
                if shift == 24:
                    mk = gmask
                    b = lax.shift_right_arithmetic(key, jnp.int32(24)) + \
                        jnp.int32(128)
                else:
                    hi = lax.shift_right_arithmetic(key, jnp.int32(shift + 8))
                    mk = jnp.logical_and(gmask, hi == prefix)
                    b = lax.bitwise_and(
                        lax.shift_right_arithmetic(key, jnp.int32(shift)), k8)
                plsc.addupdate_scatter(hist_v, [b], ones, mask=mk)
                return 0
            lax.fori_loop(0, nvcq[q], h, 0, unroll=False)
        return _scan_hist(hist_v, need)

    b1, c1, _ = level(24, None, jnp.int32(_K))
    t8 = b1 - jnp.int32(128)
    need2 = jnp.int32(_K) - c1
    b2, c2, _ = level(16, t8, need2)
    t16 = lax.bitwise_or(lax.shift_left(t8, jnp.int32(8)), b2)
    need3 = need2 - c2
    b3, c3, _ = level(8, t16, need3)
    t24 = lax.bitwise_or(lax.shift_left(t16, jnp.int32(8)), b3)
    need4 = need3 - c3
    b4, c4, h4 = level(0, t24, need4)
    thresh = lax.bitwise_or(lax.shift_left(t24, jnp.int32(8)), b4)
    need_eq = need4 - c4
    excess = h4 - need_eq

    def fixup(_):
        def cond_fn(carry):
            i, z = carry
            return jnp.logical_and(z > 0, i >= 0)

        def body_fn(carry):
            i, z = carry
            xv = row_v[pl.ds(i * _L, _L)]
            m_eq = _key_of(xv) == thresh
            eq1 = jnp.where(m_eq, ones, zeros_i)
            cnt = jnp.sum(eq1)
            pref = plsc.cumsum(eq1)
            from_end = cnt - pref + 1
            kill = jnp.logical_and(m_eq, from_end <= z)
            row_v[pl.ds(i * _L, _L)] = jnp.where(kill, neg_inf, xv)
            return i - 1, z - jnp.minimum(z, cnt)

        lax.while_loop(cond_fn, body_fn, (jnp.int32(_NV - 1), excess))
        return 0

    lax.cond(excess > 0, fixup, lambda _: 0, 0)

    def out_body(i, _):
        for u in range(_U):
            iv = i * _U + u
            xv = row_v[pl.ds(iv * _L, _L)]
            keep = _key_of(xv) >= thresh
            row_v[pl.ds(iv * _L, _L)] = jnp.where(keep, xv, zeros_f)
        return 0
    lax.fori_loop(0, _NV // _U, out_body, 0, unroll=False)


@functools.partial(
    pl.kernel,
    out_type=jax.ShapeDtypeStruct((_ROWS, _N), jnp.float32),
    mesh=plsc.VectorSubcoreMesh(core_axis_name="c", subcore_axis_name="s"),
    compiler_params=pltpu.CompilerParams(needs_layout_passes=False),
    scratch_types=[
        pltpu.VMEM((_N,), jnp.float32),
        pltpu.VMEM((_N,), jnp.float32),
        pltpu.VMEM((_NB,), jnp.int32),
    ],
)
def _topk_sc(x_hbm, out_hbm, row_v, cand_v, hist_v):
    wid = lax.axis_index("s") * _NC + lax.axis_index("c")
    for r in range(_RPW):
        row = wid * _RPW + r
        pltpu.sync_copy(x_hbm.at[row], row_v)
        _topk_row(row_v, cand_v, hist_v)
        pltpu.sync_copy(row_v, out_hbm.at[row])


@jax.jit
def kernel(x):
    return _topk_sc(x)


# speedup vs baseline: 1.3048x; 1.0460x over previous
import functools

import jax
import jax.numpy as jnp
from jax import lax
from jax.experimental import pallas as pl
from jax.experimental.pallas import tpu as pltpu
from jax.experimental.pallas import tpu_sc as plsc

_K = 64
_ROWS = 64
_N = 32768
_L = 16
_NV = _N // _L
_NQ = 4
_QV = _NV // _NQ
_QN = _N // _NQ
_UB = 4
_U = 8
_NBLK = 64
_BV = _NV // _NBLK
_NC = 2
_NS = 16
_NW = _NC * _NS
_RPW = _ROWS // _NW
_NB = 256


def _key_of(xv):
    xi = lax.bitcast_convert_type(xv, jnp.int32)
    sa = lax.shift_right_arithmetic(xi, jnp.int32(31))
    return lax.bitwise_xor(xi, lax.bitwise_and(sa, jnp.int32(0x7FFFFFFF)))


def _scan_hist(hist_v, need):
    nch = _NB // _L
    iota = lax.iota(jnp.int32, _L)

    def coarse(t, carry):
        s, jsel, ssel = carry
        for u in range(_U):
            j = (nch - 1) - (t * _U + u)
            tot = jnp.sum(hist_v[pl.ds(j * _L, _L)])
            cond = jnp.logical_and(s < need, s + tot >= need)
            jsel = jnp.where(cond, j, jsel)
            ssel = jnp.where(cond, s, ssel)
            s = s + tot
        return s, jsel, ssel

    _, jsel, ssel = lax.fori_loop(
        0, nch // _U, coarse,
        (jnp.int32(0), jnp.int32(0), jnp.int32(0)), unroll=False)

    h = hist_v[pl.ds(jsel * _L, _L)]
    rev = lax.rev(h, (0,))
    csum = plsc.cumsum(rev)
    s_lane = ssel + csum
    bin_lane = (jsel * _L + (_L - 1)) - iota
    cond = s_lane >= need
    cand_c = jnp.where(cond, lax.shift_left(bin_lane, jnp.int32(16)) +
                       (s_lane - rev), jnp.int32(-1))
    cand_h = jnp.where(cond, lax.shift_left(bin_lane, jnp.int32(16)) + rev,
                       jnp.int32(-1))
    best_c = jnp.max(cand_c)
    best_h = jnp.max(cand_h)
    bsel = lax.shift_right_arithmetic(best_c, jnp.int32(16))
    csel = lax.bitwise_and(best_c, jnp.int32(0xFFFF))
    hsel = lax.bitwise_and(best_h, jnp.int32(0xFFFF))
    return bsel, csel, hsel


def _topk_row(row_v, cand_v, hist_v):
    iota = lax.iota(jnp.int32, _L)
    ones = jnp.full((_L,), 1, jnp.int32)
    zeros_i = jnp.zeros((_L,), jnp.int32)
    zeros_f = jnp.zeros((_L,), jnp.float32)
    neg_inf = jnp.full((_L,), float("-inf"), jnp.float32)
    k8 = jnp.int32(0xFF)

    def blockmax(b, t0):
        mx = row_v[pl.ds(b * (_BV * _L), _L)]
        for u in range(1, _BV):
            mx = jnp.maximum(mx, row_v[pl.ds(b * (_BV * _L) + u * _L, _L)])
        return jnp.minimum(t0, jnp.max(mx))
    t0 = lax.fori_loop(0, _NBLK, blockmax, jnp.float32(float("inf")),
                       unroll=False)

    def filt(i, offs):
        offs = list(offs)
        for u in range(_UB):
            iv0 = i * _UB + u
            for q in range(_NQ):
                iv = q * _QV + iv0
                xv = row_v[pl.ds(iv * _L, _L)]
                m_c = xv >= t0
                plsc.store_compressed(cand_v.at[pl.ds(offs[q], _L)], xv,
                                      mask=m_c)
                pc = plsc.all_reduce_population_count(m_c)
                offs[q] = offs[q] + pc[0]
        return tuple(offs)
    offs = lax.fori_loop(
        0, _QV // _UB, filt,
        tuple(jnp.int32(q * _QN) for q in range(_NQ)), unroll=False)
    mq = [offs[q] - jnp.int32(q * _QN) for q in range(_NQ)]
    nvcq = [lax.shift_right_arithmetic(mq[q] + jnp.int32(_L - 1),
                                       jnp.int32(4)) for q in range(_NQ)]

    def level(shift, prefix, need):
        def z(i, _):
            hist_v[pl.ds(i * _L, _L)] = zeros_i
            return 0
        lax.fori_loop(0, _NB // _L, z, 0, unroll=False)

        for q in range(_NQ):
            def h(i, _, q=q):
                key = _key_of(cand_v[pl.ds(q * _QN + i * _L, _L)])
                gmask = (i * _L + iota) < mq[q]
                if shift == 24:
                    mk = gmask
                    b = lax.shift_right_arithmetic(key, jnp.int32(24)) + \
                        jnp.int32(128)
                else:
                    hi = lax.shift_right_arithmetic(key, jnp.int32(shift + 8))
                    mk = jnp.logical_and(gmask, hi == prefix)
                    b = lax.bitwise_and(
                        lax.shift_right_arithmetic(key, jnp.int32(shift)), k8)
                plsc.addupdate_scatter(hist_v, [b], ones, mask=mk)
                return 0
            lax.fori_loop(0, nvcq[q], h, 0, unroll=False)
        return _scan_hist(hist_v, need)

    b1, c1, _ = level(24, None, jnp.int32(_K))
    t8 = b1 - jnp.int32(128)
    need2 = jnp.int32(_K) - c1
    b2, c2, _ = level(16, t8, need2)
    t16 = lax.bitwise_or(lax.shift_left(t8, jnp.int32(8)), b2)
    need3 = need2 - c2
    b3, c3, _ = level(8, t16, need3)
    t24 = lax.bitwise_or(lax.shift_left(t16, jnp.int32(8)), b3)
    need4 = need3 - c3
    b4, c4, h4 = level(0, t24, need4)
    thresh = lax.bitwise_or(lax.shift_left(t24, jnp.int32(8)), b4)
    need_eq = need4 - c4
    excess = h4 - need_eq

    def fixup(_):
        def cond_fn(carry):
            i, z = carry
            return jnp.logical_and(z > 0, i >= 0)

        def body_fn(carry):
            i, z = carry
            xv = row_v[pl.ds(i * _L, _L)]
            m_eq = _key_of(xv) == thresh
            eq1 = jnp.where(m_eq, ones, zeros_i)
            cnt = jnp.sum(eq1)
            pref = plsc.cumsum(eq1)
            from_end = cnt - pref + 1
            kill = jnp.logical_and(m_eq, from_end <= z)
            row_v[pl.ds(i * _L, _L)] = jnp.where(kill, neg_inf, xv)
            return i - 1, z - jnp.minimum(z, cnt)

        lax.while_loop(cond_fn, body_fn, (jnp.int32(_NV - 1), excess))
        return 0

    lax.cond(excess > 0, fixup, lambda _: 0, 0)

    def out_body(i, _):
        for u in range(_U):
            iv = i * _U + u
            xv = row_v[pl.ds(iv * _L, _L)]
            keep = _key_of(xv) >= thresh
            row_v[pl.ds(iv * _L, _L)] = jnp.where(keep, xv, zeros_f)
        return 0
    lax.fori_loop(0, _NV // _U, out_body, 0, unroll=False)


@functools.partial(
    pl.kernel,
    out_type=jax.ShapeDtypeStruct((_ROWS, _N), jnp.float32),
    mesh=plsc.VectorSubcoreMesh(core_axis_name="c", subcore_axis_name="s"),
    compiler_params=pltpu.CompilerParams(needs_layout_passes=False),
    scratch_types=[
        pltpu.VMEM((_N,), jnp.float32),
        pltpu.VMEM((_N,), jnp.float32),
        pltpu.VMEM((_N,), jnp.float32),
        pltpu.VMEM((_NB,), jnp.int32),
        pltpu.SemaphoreType.DMA,
        pltpu.SemaphoreType.DMA,
    ],
)
def _topk_sc(x_hbm, out_hbm, row_a, row_b, cand_v, hist_v, sem_a, sem_b):
    wid = lax.axis_index("s") * _NC + lax.axis_index("c")
    r0 = wid * _RPW
    r1 = r0 + 1
    pltpu.async_copy(x_hbm.at[r0], row_a, sem_a).wait()
    in1 = pltpu.async_copy(x_hbm.at[r1], row_b, sem_b)
    _topk_row(row_a, cand_v, hist_v)
    out0 = pltpu.async_copy(row_a, out_hbm.at[r0], sem_a)
    in1.wait()
    _topk_row(row_b, cand_v, hist_v)
    out0.wait()
    pltpu.async_copy(row_b, out_hbm.at[r1], sem_b).wait()


@jax.jit
def kernel(x):
    return _topk_sc(x)


# speedup vs baseline: 1.3377x; 1.0252x over previous
import functools

import jax
import jax.numpy as jnp
from jax import lax
from jax.experimental import pallas as pl
from jax.experimental.pallas import tpu as pltpu
from jax.experimental.pallas import tpu_sc as plsc

_K = 64
_ROWS = 64
_N = 32768
_L = 16
_NV = _N // _L
_NQ = 4
_QV = _NV // _NQ
_QN = _N // _NQ
_UB = 4
_U = 8
_NBLK = 64
_BV = _NV // _NBLK
_NC = 2
_NS = 16
_NW = _NC * _NS
_RPW = _ROWS // _NW
_NB = 256


def _key_of(xv):
    xi = lax.bitcast_convert_type(xv, jnp.int32)
    sa = lax.shift_right_arithmetic(xi, jnp.int32(31))
    return lax.bitwise_xor(xi, lax.bitwise_and(sa, jnp.int32(0x7FFFFFFF)))


def _scan_hist(hist_v, need):
    nch = _NB // _L
    iota = lax.iota(jnp.int32, _L)

    def coarse(t, carry):
        s, jsel, ssel = carry
        for u in range(_U):
            j = (nch - 1) - (t * _U + u)
            tot = jnp.sum(hist_v[pl.ds(j * _L, _L)])
            cond = jnp.logical_and(s < need, s + tot >= need)
            jsel = jnp.where(cond, j, jsel)
            ssel = jnp.where(cond, s, ssel)
            s = s + tot
        return s, jsel, ssel

    _, jsel, ssel = lax.fori_loop(
        0, nch // _U, coarse,
        (jnp.int32(0), jnp.int32(0), jnp.int32(0)), unroll=False)

    h = hist_v[pl.ds(jsel * _L, _L)]
    rev = lax.rev(h, (0,))
    csum = plsc.cumsum(rev)
    s_lane = ssel + csum
    bin_lane = (jsel * _L + (_L - 1)) - iota
    cond = s_lane >= need
    cand_c = jnp.where(cond, lax.shift_left(bin_lane, jnp.int32(16)) +
                       (s_lane - rev), jnp.int32(-1))
    cand_h = jnp.where(cond, lax.shift_left(bin_lane, jnp.int32(16)) + rev,
                       jnp.int32(-1))
    best_c = jnp.max(cand_c)
    best_h = jnp.max(cand_h)
    bsel = lax.shift_right_arithmetic(best_c, jnp.int32(16))
    csel = lax.bitwise_and(best_c, jnp.int32(0xFFFF))
    hsel = lax.bitwise_and(best_h, jnp.int32(0xFFFF))
    return bsel, csel, hsel


def _topk_row(row_v, cand_v, hist_v):
    iota = lax.iota(jnp.int32, _L)
    ones = jnp.full((_L,), 1, jnp.int32)
    zeros_i = jnp.zeros((_L,), jnp.int32)
    zeros_f = jnp.zeros((_L,), jnp.float32)
    neg_inf = jnp.full((_L,), float("-inf"), jnp.float32)
    k8 = jnp.int32(0xFF)

    def blockmax(b, t0):
        mx = row_v[pl.ds(b * (_BV * _L), _L)]
        for u in range(1, _BV):
            mx = jnp.maximum(mx, row_v[pl.ds(b * (_BV * _L) + u * _L, _L)])
        return jnp.minimum(t0, jnp.max(mx))
    t0 = lax.fori_loop(0, _NBLK, blockmax, jnp.float32(float("inf")),
                       unroll=False)

    def filt(i, offs):
        offs = list(offs)
        for u in range(_UB):
            iv0 = i * _UB + u
            for q in range(_NQ):
                iv = q * _QV + iv0
                xv = row_v[pl.ds(iv * _L, _L)]
                m_c = xv >= t0
                plsc.store_compressed(cand_v.at[pl.ds(offs[q], _L)], xv,
                                      mask=m_c)
                pc = plsc.all_reduce_population_count(m_c)
                offs[q] = offs[q] + pc[0]
        return tuple(offs)
    offs = lax.fori_loop(
        0, _QV // _UB, filt,
        tuple(jnp.int32(q * _QN) for q in range(_NQ)), unroll=False)
    mq = [offs[q] - jnp.int32(q * _QN) for q in range(_NQ)]
    nvcq = [lax.shift_right_arithmetic(mq[q] + jnp.int32(_L - 1),
                                       jnp.int32(4)) for q in range(_NQ)]

    def level(shift, prefix, need):
        def z(i, _):
            hist_v[pl.ds(i * _L, _L)] = zeros_i
            return 0
        lax.fori_loop(0, _NB // _L, z, 0, unroll=False)

        for q in range(_NQ):
            def h(i, _, q=q):
                key = _key_of(cand_v[pl.ds(q * _QN + i * _L, _L)])
                gmask = (i * _L + iota) < mq[q]
                if shift == 24:
                    mk = gmask
                    b = lax.shift_right_arithmetic(key, jnp.int32(24)) + \
                        jnp.int32(128)
                else:
                    hi = lax.shift_right_arithmetic(key, jnp.int32(shift + 8))
                    mk = jnp.logical_and(gmask, hi == prefix)
                    b = lax.bitwise_and(
                        lax.shift_right_arithmetic(key, jnp.int32(shift)), k8)
                plsc.addupdate_scatter(hist_v, [b], ones, mask=mk)
                return 0
            lax.fori_loop(0, nvcq[q], h, 0, unroll=False)
        return _scan_hist(hist_v, need)

    b1, c1, _ = level(24, None, jnp.int32(_K))
    t8 = b1 - jnp.int32(128)
    need2 = jnp.int32(_K) - c1
    b2, c2, _ = level(16, t8, need2)
    t16 = lax.bitwise_or(lax.shift_left(t8, jnp.int32(8)), b2)
    need3 = need2 - c2
    b3, c3, _ = level(8, t16, need3)
    t24 = lax.bitwise_or(lax.shift_left(t16, jnp.int32(8)), b3)
    need4 = need3 - c3
    b4, c4, h4 = level(0, t24, need4)
    thresh = lax.bitwise_or(lax.shift_left(t24, jnp.int32(8)), b4)
    need_eq = need4 - c4
    excess = h4 - need_eq

    def fixup(_):
        def cond_fn(carry):
            i, z = carry
            return jnp.logical_and(z > 0, i >= 0)

        def body_fn(carry):
            i, z = carry
            xv = row_v[pl.ds(i * _L, _L)]
            m_eq = _key_of(xv) == thresh
            eq1 = jnp.where(m_eq, ones, zeros_i)
            cnt = jnp.sum(eq1)
            pref = plsc.cumsum(eq1)
            from_end = cnt - pref + 1
            kill = jnp.logical_and(m_eq, from_end <= z)
            row_v[pl.ds(i * _L, _L)] = jnp.where(kill, neg_inf, xv)
            return i - 1, z - jnp.minimum(z, cnt)

        lax.while_loop(cond_fn, body_fn, (jnp.int32(_NV - 1), excess))
        return 0

    lax.cond(excess > 0, fixup, lambda _: 0, 0)

    tkv = zeros_i + thresh
    sak = lax.shift_right_arithmetic(tkv, jnp.int32(31))
    tfv = lax.bitcast_convert_type(
        lax.bitwise_xor(tkv, lax.bitwise_and(sak, jnp.int32(0x7FFFFFFF))),
        jnp.float32)

    def out_body(i, _):
        for u in range(2 * _U):
            iv = i * (2 * _U) + u
            xv = row_v[pl.ds(iv * _L, _L)]
            keep = xv >= tfv
            row_v[pl.ds(iv * _L, _L)] = jnp.where(keep, xv, zeros_f)
        return 0
    lax.fori_loop(0, _NV // (2 * _U), out_body, 0, unroll=False)


@functools.partial(
    pl.kernel,
    out_type=jax.ShapeDtypeStruct((_ROWS, _N), jnp.float32),
    mesh=plsc.VectorSubcoreMesh(core_axis_name="c", subcore_axis_name="s"),
    compiler_params=pltpu.CompilerParams(needs_layout_passes=False),
    scratch_types=[
        pltpu.VMEM((_N,), jnp.float32),
        pltpu.VMEM((_N,), jnp.float32),
        pltpu.VMEM((_N,), jnp.float32),
        pltpu.VMEM((_NB,), jnp.int32),
        pltpu.SemaphoreType.DMA,
        pltpu.SemaphoreType.DMA,
    ],
)
def _topk_sc(x_hbm, out_hbm, row_a, row_b, cand_v, hist_v, sem_a, sem_b):
    wid = lax.axis_index("s") * _NC + lax.axis_index("c")
    r0 = wid * _RPW
    r1 = r0 + 1
    pltpu.async_copy(x_hbm.at[r0], row_a, sem_a).wait()
    in1 = pltpu.async_copy(x_hbm.at[r1], row_b, sem_b)
    _topk_row(row_a, cand_v, hist_v)
    out0 = pltpu.async_copy(row_a, out_hbm.at[r0], sem_a)
    in1.wait()
    _topk_row(row_b, cand_v, hist_v)
    out0.wait()
    pltpu.async_copy(row_b, out_hbm.at[r1], sem_b).wait()


@jax.jit
def kernel(x):
    return _topk_sc(x)
